# baseline (speedup 1.0000x reference)
import jax
import jax.numpy as jnp
from jax.experimental import pallas as pl
from jax.experimental.pallas import tpu as pltpu
from jax.experimental.pallas import tpu_sc as plsc

N_ROWS = 8192
NUM_FEATS = 4096

CHUNK = 128
N_CHUNKS = N_ROWS // CHUNK
NSPLIT = 4
KPC = N_CHUNKS // NSPLIT
TABLE_ROWS_C = KPC * NUM_FEATS

GATHER_WINDOW = 128


def _transpose_body(x_ref, o_ref):
    o_ref[...] = x_ref[...].T


def _transpose_fwd(x, c):
    return pl.pallas_call(
        _transpose_body,
        grid=(KPC,),
        in_specs=[pl.BlockSpec((CHUNK, NUM_FEATS),
                               lambda k, c=c: (c * KPC + k, 0))],
        out_specs=pl.BlockSpec((NUM_FEATS, CHUNK), lambda k: (k, 0)),
        out_shape=jax.ShapeDtypeStruct((TABLE_ROWS_C, CHUNK), x.dtype),
        compiler_params=pltpu.CompilerParams(
            dimension_semantics=("parallel",),
        ),
    )(x)


def _transpose_bwd_first(g):
    return pl.pallas_call(
        _transpose_body,
        grid=(KPC,),
        in_specs=[pl.BlockSpec((NUM_FEATS, CHUNK), lambda k: (k, 0))],
        out_specs=pl.BlockSpec((CHUNK, NUM_FEATS), lambda k: (k, 0)),
        out_shape=jax.ShapeDtypeStruct((N_ROWS, NUM_FEATS), g.dtype),
        compiler_params=pltpu.CompilerParams(
            dimension_semantics=("parallel",),
        ),
    )(g)


def _transpose_bwd_next(g, prev, c):
    def body(g_ref, prev_ref, o_ref):
        del prev_ref
        o_ref[...] = g_ref[...].T

    return pl.pallas_call(
        body,
        grid=(KPC,),
        in_specs=[
            pl.BlockSpec((NUM_FEATS, CHUNK), lambda k: (k, 0)),
            pl.BlockSpec(memory_space=pl.ANY),
        ],
        out_specs=pl.BlockSpec((CHUNK, NUM_FEATS),
                               lambda k, c=c: (c * KPC + k, 0)),
        out_shape=jax.ShapeDtypeStruct((N_ROWS, NUM_FEATS), g.dtype),
        input_output_aliases={1: 0},
        compiler_params=pltpu.CompilerParams(
            dimension_semantics=("parallel",),
        ),
    )(g, prev)


def _sc_gather(table, idx3):
    vector_mesh = plsc.VectorSubcoreMesh(
        core_axis_name="core", subcore_axis_name="subcore")
    n_windows = TABLE_ROWS_C // GATHER_WINDOW

    @pl.kernel(out_type=jax.ShapeDtypeStruct(table.shape, table.dtype),
               mesh=vector_mesh)
    def k(x_hbm, i_hbm, o_hbm):
        def body(i_vmem, o_vmem):
            pltpu.sync_copy(x_hbm.at[i_vmem.at[0, 0]], o_vmem)

        pltpu.emit_pipeline(
            body,
            grid=(n_windows,),
            in_specs=[pl.BlockSpec((1, 1, GATHER_WINDOW),
                                   index_map=lambda i: (i, 0, 0))],
            out_specs=[pl.BlockSpec((GATHER_WINDOW, CHUNK),
                                    index_map=lambda i: (i, 0))],
            core_axis_name=("core", "subcore"),
            dimension_semantics=(pltpu.PARALLEL,),
        )(i_hbm, o_hbm)

    return k(table, idx3)


def kernel(x, indices):
    idx = indices.astype(jnp.int32)
    idx3 = (jnp.arange(KPC, dtype=jnp.int32)[:, None] * NUM_FEATS
            + idx[None, :]).reshape(TABLE_ROWS_C // GATHER_WINDOW, 1,
                                    GATHER_WINDOW)
    gts = []
    for c in range(NSPLIT):
        xt_c = _transpose_fwd(x, c)
        gts.append(_sc_gather(xt_c, idx3))
    out = _transpose_bwd_first(gts[0])
    for c in range(1, NSPLIT):
        out = _transpose_bwd_next(gts[c], out, c)
    return out


# speedup vs baseline: 1.0656x; 1.0656x over previous
import jax
import jax.numpy as jnp
from jax.experimental import pallas as pl
from jax.experimental.pallas import tpu as pltpu
from jax.experimental.pallas import tpu_sc as plsc

N_ROWS = 8192
NUM_FEATS = 4096

CHUNK = 256
N_CHUNKS = N_ROWS // CHUNK
NSPLIT = 2
KPC = N_CHUNKS // NSPLIT
TABLE_ROWS_C = KPC * NUM_FEATS

GATHER_WINDOW = 128


def _transpose_body(x_ref, o_ref):
    o_ref[...] = x_ref[...].T


def _transpose_fwd(x, c):
    return pl.pallas_call(
        _transpose_body,
        grid=(KPC,),
        in_specs=[pl.BlockSpec((CHUNK, NUM_FEATS),
                               lambda k, c=c: (c * KPC + k, 0))],
        out_specs=pl.BlockSpec((NUM_FEATS, CHUNK), lambda k: (k, 0)),
        out_shape=jax.ShapeDtypeStruct((TABLE_ROWS_C, CHUNK), x.dtype),
        compiler_params=pltpu.CompilerParams(
            dimension_semantics=("parallel",),
        ),
    )(x)


def _transpose_bwd_first(g):
    return pl.pallas_call(
        _transpose_body,
        grid=(KPC,),
        in_specs=[pl.BlockSpec((NUM_FEATS, CHUNK), lambda k: (k, 0))],
        out_specs=pl.BlockSpec((CHUNK, NUM_FEATS), lambda k: (k, 0)),
        out_shape=jax.ShapeDtypeStruct((N_ROWS, NUM_FEATS), g.dtype),
        compiler_params=pltpu.CompilerParams(
            dimension_semantics=("parallel",),
        ),
    )(g)


def _transpose_bwd_next(g, prev, c):
    def body(g_ref, prev_ref, o_ref):
        del prev_ref
        o_ref[...] = g_ref[...].T

    return pl.pallas_call(
        body,
        grid=(KPC,),
        in_specs=[
            pl.BlockSpec((NUM_FEATS, CHUNK), lambda k: (k, 0)),
            pl.BlockSpec(memory_space=pl.ANY),
        ],
        out_specs=pl.BlockSpec((CHUNK, NUM_FEATS),
                               lambda k, c=c: (c * KPC + k, 0)),
        out_shape=jax.ShapeDtypeStruct((N_ROWS, NUM_FEATS), g.dtype),
        input_output_aliases={1: 0},
        compiler_params=pltpu.CompilerParams(
            dimension_semantics=("parallel",),
        ),
    )(g, prev)


def _sc_gather(table, idx3):
    vector_mesh = plsc.VectorSubcoreMesh(
        core_axis_name="core", subcore_axis_name="subcore")
    n_windows = TABLE_ROWS_C // GATHER_WINDOW

    @pl.kernel(out_type=jax.ShapeDtypeStruct(table.shape, table.dtype),
               mesh=vector_mesh)
    def k(x_hbm, i_hbm, o_hbm):
        def body(i_vmem, o_vmem):
            pltpu.sync_copy(x_hbm.at[i_vmem.at[0, 0]], o_vmem)

        pltpu.emit_pipeline(
            body,
            grid=(n_windows,),
            in_specs=[pl.BlockSpec((1, 1, GATHER_WINDOW),
                                   index_map=lambda i: (i, 0, 0))],
            out_specs=[pl.BlockSpec((GATHER_WINDOW, CHUNK),
                                    index_map=lambda i: (i, 0))],
            core_axis_name=("core", "subcore"),
            dimension_semantics=(pltpu.PARALLEL,),
        )(i_hbm, o_hbm)

    return k(table, idx3)


def kernel(x, indices):
    idx = indices.astype(jnp.int32)
    idx3 = (jnp.arange(KPC, dtype=jnp.int32)[:, None] * NUM_FEATS
            + idx[None, :]).reshape(TABLE_ROWS_C // GATHER_WINDOW, 1,
                                    GATHER_WINDOW)
    gts = []
    for c in range(NSPLIT):
        xt_c = _transpose_fwd(x, c)
        gts.append(_sc_gather(xt_c, idx3))
    out = _transpose_bwd_first(gts[0])
    for c in range(1, NSPLIT):
        out = _transpose_bwd_next(gts[c], out, c)
    return out
